# Initial kernel scaffold; baseline (speedup 1.0000x reference)
import jax
import jax.numpy as jnp
from jax.experimental import pallas as pl


def kernel(x, edge_index, W1, b1, W2, b2):
    raise NotImplementedError("write your pallas kernel here")


# speedup vs baseline: 30.8098x; 30.8098x over previous
import functools

import jax
import jax.numpy as jnp
from jax import lax
from jax.experimental import pallas as pl
from jax.experimental.pallas import tpu as pltpu
from jax.experimental.pallas import tpu_sc as plsc

NUM_CORES = 2
NUM_SUB = 16
NUM_TILES = NUM_CORES * NUM_SUB
LANES = 16
CHUNK = 512


def _sc_mesh():
    return plsc.VectorSubcoreMesh(core_axis_name="c", subcore_axis_name="s")


_SC_PARAMS = pltpu.CompilerParams(
    use_tc_tiling_on_sc=False, needs_layout_passes=False
)


def _rsqrt16(x):
    i = plsc.bitcast(x, jnp.int32)
    y = plsc.bitcast(jnp.int32(0x5F3759DF) - jnp.right_shift(i, 1),
                     jnp.float32)
    for _ in range(3):
        y = y * (1.5 - 0.5 * x * y * y)
    return y


def _sc_degree(dst3, zeros_tbl, ones_rows):
    ch = dst3.shape[1]
    n_pad = zeros_tbl.shape[0]
    rows_per_sub = n_pad // NUM_SUB

    @functools.partial(
        pl.kernel,
        out_type=jax.ShapeDtypeStruct((NUM_CORES, n_pad, LANES), jnp.float32),
        mesh=_sc_mesh(),
        compiler_params=_SC_PARAMS,
        scratch_types=[
            pltpu.VMEM((ch, CHUNK), jnp.int32),
            pltpu.VMEM((CHUNK, LANES), jnp.float32),
            pltpu.VMEM_SHARED((n_pad, LANES), jnp.float32),
        ],
    )
    def k(dst_h, z_h, ones_h, out_h, di_v, ones_v, acc_sh):
        c = lax.axis_index("c")
        s = lax.axis_index("s")
        g = c * NUM_SUB + s
        sl = pl.ds(s * rows_per_sub, rows_per_sub)
        pltpu.sync_copy(z_h.at[sl], acc_sh.at[sl])
        pltpu.sync_copy(dst_h.at[g], di_v)
        pltpu.sync_copy(ones_h, ones_v)
        plsc.subcore_barrier()

        @pl.loop(0, ch)
        def _(j):
            pltpu.sync_copy(ones_v, acc_sh.at[di_v.at[j]], add=True)

        plsc.subcore_barrier()
        pltpu.sync_copy(acc_sh.at[sl], out_h.at[c, sl])

    return k(dst3, zeros_tbl, ones_rows)


def _sc_aggregate(src3, dst3, degp, feat, zeros_tbl, prevp=None, b1r=None):
    ch = src3.shape[1]
    n_pad = degp.shape[1]
    rows_per_sub = n_pad // NUM_SUB
    first_layer = prevp is None

    extra_in = () if first_layer else (prevp, b1r)
    extra_scratch = () if first_layer else (
        pltpu.VMEM((rows_per_sub, LANES), jnp.float32),
        pltpu.VMEM((rows_per_sub, LANES), jnp.float32),
        pltpu.VMEM((LANES,), jnp.float32),
    )

    @functools.partial(
        pl.kernel,
        out_type=(
            jax.ShapeDtypeStruct((NUM_CORES, n_pad, LANES), jnp.float32),
            jax.ShapeDtypeStruct((n_pad, LANES), jnp.float32),
        ),
        mesh=_sc_mesh(),
        compiler_params=_SC_PARAMS,
        scratch_types=[
            pltpu.VMEM((ch, CHUNK), jnp.int32),
            pltpu.VMEM((ch, CHUNK), jnp.int32),
            pltpu.VMEM((CHUNK, LANES), jnp.float32),
            pltpu.VMEM((CHUNK, LANES), jnp.float32),
            pltpu.VMEM((rows_per_sub, LANES), jnp.float32),
            pltpu.VMEM((rows_per_sub, LANES), jnp.float32),
            pltpu.VMEM((rows_per_sub, LANES), jnp.float32),
            *extra_scratch,
            pltpu.VMEM_SHARED((n_pad, LANES), jnp.float32),
            pltpu.SemaphoreType.DMA,
            pltpu.SemaphoreType.DMA,
        ],
    )
    def k(src_h, dst_h, degp_h, feat_h, z_h, *rest):
        if first_layer:
            (out_h, tbl_h, si_v, di_v, rows0, rows1, d0_v, d1_v, f_v,
             acc_sh, sem0, sem1) = rest
            p0_v = p1_v = b_v = prevp_h = b1_h = None
        else:
            (prevp_h, b1_h, out_h, tbl_h, si_v, di_v, rows0, rows1, d0_v,
             d1_v, f_v, p0_v, p1_v, b_v, acc_sh, sem0, sem1) = rest
        c = lax.axis_index("c")
        s = lax.axis_index("s")
        g = c * NUM_SUB + s
        sl = pl.ds(s * rows_per_sub, rows_per_sub)

        pltpu.sync_copy(z_h.at[sl], acc_sh.at[sl])
        pltpu.sync_copy(src_h.at[g], si_v)
        pltpu.sync_copy(dst_h.at[g], di_v)
        pltpu.sync_copy(degp_h.at[0, sl], d0_v)
        pltpu.sync_copy(degp_h.at[1, sl], d1_v)
        pltpu.sync_copy(feat_h.at[sl], f_v)
        if not first_layer:
            pltpu.sync_copy(prevp_h.at[0, sl], p0_v)
            pltpu.sync_copy(prevp_h.at[1, sl], p1_v)
            pltpu.sync_copy(b1_h, b_v)

        @pl.loop(0, rows_per_sub)
        def _(r):
            deg = d0_v[r] + d1_v[r] + 1.0
            dis = _rsqrt16(deg)
            if first_layer:
                val = f_v[r] * dis
            else:
                agg = p0_v[r] + p1_v[r] + f_v[r]
                z = agg * dis + b_v[...]
                val = jnp.maximum(z, 0.0) * dis
            f_v[r] = val

        pltpu.sync_copy(f_v, tbl_h.at[sl])
        plsc.subcore_barrier()

        pltpu.async_copy(tbl_h.at[si_v.at[0]], rows0, sem0)
        pltpu.async_copy(tbl_h.at[si_v.at[1]], rows1, sem1)

        @pl.loop(0, ch, step=2)
        def _(j):
            pltpu.make_async_copy(tbl_h.at[si_v.at[j]], rows0, sem0).wait()
            pltpu.sync_copy(rows0, acc_sh.at[di_v.at[j]], add=True)

            @pl.when(j + 2 < ch)
            def _():
                pltpu.async_copy(tbl_h.at[si_v.at[j + 2]], rows0, sem0)

            pltpu.make_async_copy(tbl_h.at[si_v.at[j + 1]], rows1, sem1).wait()
            pltpu.sync_copy(rows1, acc_sh.at[di_v.at[j + 1]], add=True)

            @pl.when(j + 3 < ch)
            def _():
                pltpu.async_copy(tbl_h.at[si_v.at[j + 3]], rows1, sem1)

        plsc.subcore_barrier()
        pltpu.sync_copy(acc_sh.at[sl], out_h.at[c, sl])

    return k(src3, dst3, degp, feat, zeros_tbl, *extra_in)


def _tc_mm1(x, w1, n_pad):
    n = x.shape[0]
    d_in = x.shape[1]
    blk = n_pad // NUM_SUB

    def body(x_ref, w_ref, h_ref):
        h_ref[...] = jnp.dot(x_ref[...], w_ref[...],
                             preferred_element_type=jnp.float32)

    return pl.pallas_call(
        body,
        grid=(n_pad // blk,),
        in_specs=[
            pl.BlockSpec((blk, d_in), lambda i: (i, 0)),
            pl.BlockSpec((d_in, LANES), lambda i: (0, 0)),
        ],
        out_specs=pl.BlockSpec((blk, LANES), lambda i: (i, 0)),
        out_shape=jax.ShapeDtypeStruct((n_pad, LANES), jnp.float32),
    )(x, w1)


def _tc_post(degp, aggp, tbl, w2, b2r, n):
    d_out = w2.shape[1]
    blk = 400
    n_pad = tbl.shape[0]

    def body(d_ref, a_ref, t_ref, w_ref, b_ref, o_ref):
        deg = d_ref[0] + d_ref[1] + 1.0
        dis = lax.rsqrt(deg)
        gsc = (a_ref[0] + a_ref[1] + t_ref[...]) * dis
        o_ref[...] = (
            jnp.dot(gsc, w_ref[...], preferred_element_type=jnp.float32)
            + b_ref[...]
        )

    return pl.pallas_call(
        body,
        grid=(n // blk,),
        in_specs=[
            pl.BlockSpec((NUM_CORES, blk, LANES), lambda i: (0, i, 0)),
            pl.BlockSpec((NUM_CORES, blk, LANES), lambda i: (0, i, 0)),
            pl.BlockSpec((blk, LANES), lambda i: (i, 0)),
            pl.BlockSpec((LANES, d_out), lambda i: (0, 0)),
            pl.BlockSpec((1, d_out), lambda i: (0, 0)),
        ],
        out_specs=pl.BlockSpec((blk, d_out), lambda i: (i, 0)),
        out_shape=jax.ShapeDtypeStruct((n, d_out), jnp.float32),
    )(degp, aggp, tbl, w2, b2r)


def kernel(x, edge_index, W1, b1, W2, b2):
    n, _ = x.shape
    e = edge_index.shape[1]

    n_pad = ((n + NUM_TILES * NUM_SUB - 1) // (NUM_TILES * NUM_SUB)) * (
        NUM_TILES * NUM_SUB
    )
    e_tile = ((e + NUM_TILES * CHUNK - 1) // (NUM_TILES * CHUNK)) * CHUNK
    e_pad = e_tile * NUM_TILES
    ch = e_tile // CHUNK

    ei3 = jnp.pad(
        edge_index, ((0, 0), (0, e_pad - e)), constant_values=n_pad - 1
    ).reshape(2, NUM_TILES, ch, CHUNK)
    src3 = ei3[0]
    dst3 = ei3[1]

    zeros_tbl = jnp.zeros((n_pad, LANES), jnp.float32)
    ones_rows = jnp.ones((CHUNK, LANES), jnp.float32)
    b1r = b1

    degp = _sc_degree(dst3, zeros_tbl, ones_rows)
    h1 = _tc_mm1(x, W1, n_pad)
    aggp1, tbl1 = _sc_aggregate(src3, dst3, degp, h1, zeros_tbl)
    aggp2, tbl2 = _sc_aggregate(
        src3, dst3, degp, tbl1, zeros_tbl, prevp=aggp1, b1r=b1r
    )
    out = _tc_post(degp, aggp2, tbl2, W2, b2.reshape(1, -1), n)
    return out


# speedup vs baseline: 35.0808x; 1.1386x over previous
import functools

import jax
import jax.numpy as jnp
from jax import lax
from jax.experimental import pallas as pl
from jax.experimental.pallas import tpu as pltpu
from jax.experimental.pallas import tpu_sc as plsc

NUM_CORES = 2
NUM_SUB = 16
NUM_TILES = NUM_CORES * NUM_SUB
LANES = 16
CHUNK = 512
CH0 = 13
CH1 = 7


def _sc_mesh():
    return plsc.VectorSubcoreMesh(core_axis_name="c", subcore_axis_name="s")


_SC_PARAMS = pltpu.CompilerParams(
    use_tc_tiling_on_sc=False, needs_layout_passes=False
)


def _rsqrt16(x):
    i = plsc.bitcast(x, jnp.int32)
    y = plsc.bitcast(jnp.int32(0x5F3759DF) - jnp.right_shift(i, 1),
                     jnp.float32)
    for _ in range(3):
        y = y * (1.5 - 0.5 * x * y * y)
    return y


def _sc_degree(dst2, zeros_tbl, ones_rows):
    n_pad = zeros_tbl.shape[0]
    rows_per_sub = n_pad // NUM_SUB

    @functools.partial(
        pl.kernel,
        out_type=jax.ShapeDtypeStruct((NUM_CORES, n_pad, LANES), jnp.float32),
        mesh=_sc_mesh(),
        compiler_params=_SC_PARAMS,
        scratch_types=[
            pltpu.VMEM((max(CH0, CH1), CHUNK), jnp.int32),
            pltpu.VMEM((CHUNK, LANES), jnp.float32),
            pltpu.VMEM_SHARED((n_pad, LANES), jnp.float32),
        ],
    )
    def k(dst_h, z_h, ones_h, out_h, di_v, ones_v, acc_sh):
        c = lax.axis_index("c")
        s = lax.axis_index("s")
        sl = pl.ds(s * rows_per_sub, rows_per_sub)
        pltpu.sync_copy(z_h.at[sl], acc_sh.at[sl])
        pltpu.sync_copy(ones_h, ones_v)
        plsc.subcore_barrier()

        def scatter_ones(ch_n, base):
            pltpu.sync_copy(dst_h.at[pl.ds(base, ch_n)],
                            di_v.at[pl.ds(0, ch_n)])

            @pl.loop(0, ch_n)
            def _(j):
                pltpu.sync_copy(ones_v, acc_sh.at[di_v.at[j]], add=True)

        @pl.when(c == 0)
        def _():
            scatter_ones(CH0, s * CH0)

        @pl.when(c == 1)
        def _():
            scatter_ones(CH1, NUM_SUB * CH0 + s * CH1)

        plsc.subcore_barrier()
        pltpu.sync_copy(acc_sh.at[sl], out_h.at[c, sl])

    return k(dst2, zeros_tbl, ones_rows)


def _sc_aggregate(src2, dst2, degp, feat, zeros_tbl, prevp=None, b1r=None):
    n_pad = degp.shape[1]
    rows_per_sub = n_pad // NUM_SUB
    first_layer = prevp is None

    extra_in = () if first_layer else (prevp, b1r)
    extra_scratch = () if first_layer else (
        pltpu.VMEM((rows_per_sub, LANES), jnp.float32),
        pltpu.VMEM((rows_per_sub, LANES), jnp.float32),
        pltpu.VMEM((LANES,), jnp.float32),
    )

    @functools.partial(
        pl.kernel,
        out_type=(
            jax.ShapeDtypeStruct((NUM_CORES, n_pad, LANES), jnp.float32),
            jax.ShapeDtypeStruct((n_pad, LANES), jnp.float32),
        ),
        mesh=_sc_mesh(),
        compiler_params=_SC_PARAMS,
        scratch_types=[
            pltpu.VMEM((max(CH0, CH1), CHUNK), jnp.int32),
            pltpu.VMEM((max(CH0, CH1), CHUNK), jnp.int32),
            pltpu.VMEM((CHUNK, LANES), jnp.float32),
            pltpu.VMEM((CHUNK, LANES), jnp.float32),
            pltpu.VMEM((rows_per_sub, LANES), jnp.float32),
            pltpu.VMEM((rows_per_sub, LANES), jnp.float32),
            pltpu.VMEM((rows_per_sub, LANES), jnp.float32),
            *extra_scratch,
            pltpu.VMEM_SHARED((n_pad, LANES), jnp.float32),
            pltpu.SemaphoreType.DMA,
            pltpu.SemaphoreType.DMA,
        ],
    )
    def k(src_h, dst_h, degp_h, feat_h, z_h, *rest):
        if first_layer:
            (out_h, tbl_h, si_v, di_v, rows0, rows1, d0_v, d1_v, f_v,
             acc_sh, sem0, sem1) = rest
            p0_v = p1_v = b_v = prevp_h = b1_h = None
        else:
            (prevp_h, b1_h, out_h, tbl_h, si_v, di_v, rows0, rows1, d0_v,
             d1_v, f_v, p0_v, p1_v, b_v, acc_sh, sem0, sem1) = rest
        c = lax.axis_index("c")
        s = lax.axis_index("s")
        sl = pl.ds(s * rows_per_sub, rows_per_sub)

        pltpu.sync_copy(z_h.at[sl], acc_sh.at[sl])
        pltpu.sync_copy(degp_h.at[0, sl], d0_v)
        pltpu.sync_copy(degp_h.at[1, sl], d1_v)
        pltpu.sync_copy(feat_h.at[sl], f_v)
        if not first_layer:
            pltpu.sync_copy(prevp_h.at[0, sl], p0_v)
            pltpu.sync_copy(prevp_h.at[1, sl], p1_v)
            pltpu.sync_copy(b1_h, b_v)

        @pl.loop(0, rows_per_sub)
        def _(r):
            deg = d0_v[r] + d1_v[r] + 1.0
            dis = _rsqrt16(deg)
            if first_layer:
                val = f_v[r] * dis
            else:
                agg = p0_v[r] + p1_v[r] + f_v[r]
                z = agg * dis + b_v[...]
                val = jnp.maximum(z, 0.0) * dis
            f_v[r] = val

        pltpu.sync_copy(f_v, tbl_h.at[sl])
        plsc.subcore_barrier()

        def run_edges(ch_n, base):
            pltpu.sync_copy(src_h.at[pl.ds(base, ch_n)],
                            si_v.at[pl.ds(0, ch_n)])
            pltpu.sync_copy(dst_h.at[pl.ds(base, ch_n)],
                            di_v.at[pl.ds(0, ch_n)])

            pltpu.async_copy(tbl_h.at[si_v.at[0]], rows0, sem0)

            @pl.when(ch_n > 1)
            def _():
                pltpu.async_copy(tbl_h.at[si_v.at[1]], rows1, sem1)

            @pl.loop(0, ch_n, step=2)
            def _(j):
                pltpu.make_async_copy(tbl_h.at[si_v.at[j]], rows0,
                                      sem0).wait()
                pltpu.sync_copy(rows0, acc_sh.at[di_v.at[j]], add=True)

                @pl.when(j + 2 < ch_n)
                def _():
                    pltpu.async_copy(tbl_h.at[si_v.at[j + 2]], rows0, sem0)

                @pl.when(j + 1 < ch_n)
                def _():
                    pltpu.make_async_copy(tbl_h.at[si_v.at[j + 1]], rows1,
                                          sem1).wait()
                    pltpu.sync_copy(rows1, acc_sh.at[di_v.at[j + 1]],
                                    add=True)

                    @pl.when(j + 3 < ch_n)
                    def _():
                        pltpu.async_copy(tbl_h.at[si_v.at[j + 3]], rows1,
                                         sem1)

        @pl.when(c == 0)
        def _():
            run_edges(CH0, s * CH0)

        @pl.when(c == 1)
        def _():
            run_edges(CH1, NUM_SUB * CH0 + s * CH1)

        plsc.subcore_barrier()
        pltpu.sync_copy(acc_sh.at[sl], out_h.at[c, sl])

    return k(src2, dst2, degp, feat, zeros_tbl, *extra_in)


def _tc_mm1(x, w1, n_pad):
    d_in = x.shape[1]
    blk = 2560

    def body(x_ref, w_ref, h_ref):
        h_ref[...] = jnp.dot(x_ref[...], w_ref[...],
                             preferred_element_type=jnp.float32)

    return pl.pallas_call(
        body,
        grid=(n_pad // blk,),
        in_specs=[
            pl.BlockSpec((blk, d_in), lambda i: (i, 0)),
            pl.BlockSpec((d_in, LANES), lambda i: (0, 0)),
        ],
        out_specs=pl.BlockSpec((blk, LANES), lambda i: (i, 0)),
        out_shape=jax.ShapeDtypeStruct((n_pad, LANES), jnp.float32),
    )(x, w1)


def _tc_post(degp, aggp, tbl, w2, b2r, n):
    d_out = w2.shape[1]
    blk = 2000

    def body(d_ref, a_ref, t_ref, w_ref, b_ref, o_ref):
        deg = d_ref[0] + d_ref[1] + 1.0
        dis = lax.rsqrt(deg)
        gsc = (a_ref[0] + a_ref[1] + t_ref[...]) * dis
        o_ref[...] = (
            jnp.dot(gsc, w_ref[...], preferred_element_type=jnp.float32)
            + b_ref[...]
        )

    return pl.pallas_call(
        body,
        grid=(n // blk,),
        in_specs=[
            pl.BlockSpec((NUM_CORES, blk, LANES), lambda i: (0, i, 0)),
            pl.BlockSpec((NUM_CORES, blk, LANES), lambda i: (0, i, 0)),
            pl.BlockSpec((blk, LANES), lambda i: (i, 0)),
            pl.BlockSpec((LANES, d_out), lambda i: (0, 0)),
            pl.BlockSpec((1, d_out), lambda i: (0, 0)),
        ],
        out_specs=pl.BlockSpec((blk, d_out), lambda i: (i, 0)),
        out_shape=jax.ShapeDtypeStruct((n, d_out), jnp.float32),
    )(degp, aggp, tbl, w2, b2r)


def kernel(x, edge_index, W1, b1, W2, b2):
    n, _ = x.shape
    e = edge_index.shape[1]

    n_pad = ((n + NUM_TILES * NUM_SUB - 1) // (NUM_TILES * NUM_SUB)) * (
        NUM_TILES * NUM_SUB
    )
    per_pair = (CH0 + CH1) * CHUNK
    e_pad = ((e + NUM_SUB * per_pair - 1) // (NUM_SUB * per_pair)) * (
        NUM_SUB * per_pair
    )
    n_chunks = e_pad // CHUNK

    ei2 = jnp.pad(
        edge_index, ((0, 0), (0, e_pad - e)), constant_values=n_pad - 1
    ).reshape(2, n_chunks, CHUNK)
    src2 = ei2[0]
    dst2 = ei2[1]

    zeros_tbl = jnp.zeros((n_pad, LANES), jnp.float32)
    ones_rows = jnp.ones((CHUNK, LANES), jnp.float32)

    degp = _sc_degree(dst2, zeros_tbl, ones_rows)
    h1 = _tc_mm1(x, W1, n_pad)
    aggp1, tbl1 = _sc_aggregate(src2, dst2, degp, h1, zeros_tbl)
    aggp2, tbl2 = _sc_aggregate(
        src2, dst2, degp, tbl1, zeros_tbl, prevp=aggp1, b1r=b1
    )
    out = _tc_post(degp, aggp2, tbl2, W2, b2.reshape(1, -1), n)
    return out


# speedup vs baseline: 35.7603x; 1.0194x over previous
import functools

import jax
import jax.numpy as jnp
from jax import lax
from jax.experimental import pallas as pl
from jax.experimental.pallas import tpu as pltpu
from jax.experimental.pallas import tpu_sc as plsc

NUM_CORES = 2
NUM_SUB = 16
NUM_TILES = NUM_CORES * NUM_SUB
LANES = 16
CHUNK = 512
CH0 = 15
CH1 = 5
RB0 = 480
RB1 = 160


def _sc_mesh():
    return plsc.VectorSubcoreMesh(core_axis_name="c", subcore_axis_name="s")


_SC_PARAMS = pltpu.CompilerParams(
    use_tc_tiling_on_sc=False, needs_layout_passes=False
)


def _rsqrt16(x):
    i = plsc.bitcast(x, jnp.int32)
    y = plsc.bitcast(jnp.int32(0x5F3759DF) - jnp.right_shift(i, 1),
                     jnp.float32)
    for _ in range(3):
        y = y * (1.5 - 0.5 * x * y * y)
    return y


def _core_split(c, fn, n0, n1):
    @pl.when(c == 0)
    def _():
        fn(n0, lax.axis_index("s") * n0)

    @pl.when(c == 1)
    def _():
        fn(n1, NUM_SUB * n0 + lax.axis_index("s") * n1)


def _sc_degree(dst2, ones_rows, n_pad):
    rows_per_sub = n_pad // NUM_SUB

    @functools.partial(
        pl.kernel,
        out_type=jax.ShapeDtypeStruct((NUM_CORES, n_pad, LANES), jnp.float32),
        mesh=_sc_mesh(),
        compiler_params=_SC_PARAMS,
        scratch_types=[
            pltpu.VMEM((max(CH0, CH1), CHUNK), jnp.int32),
            pltpu.VMEM((rows_per_sub, LANES), jnp.float32),
            pltpu.VMEM_SHARED((n_pad, LANES), jnp.float32),
        ],
    )
    def k(dst_h, ones_h, out_h, di_v, ones_v, acc_sh):
        c = lax.axis_index("c")
        s = lax.axis_index("s")
        sl = pl.ds(s * rows_per_sub, rows_per_sub)
        pltpu.sync_copy(ones_h, ones_v)
        pltpu.sync_copy(ones_v, acc_sh.at[sl])
        plsc.subcore_barrier()

        def scatter_ones(ch_n, base):
            pltpu.sync_copy(dst_h.at[pl.ds(base, ch_n)],
                            di_v.at[pl.ds(0, ch_n)])

            @pl.loop(0, ch_n)
            def _(j):
                pltpu.sync_copy(ones_v.at[pl.ds(0, CHUNK)],
                                acc_sh.at[di_v.at[j]], add=True)

        _core_split(c, scatter_ones, CH0, CH1)

        plsc.subcore_barrier()
        pltpu.sync_copy(acc_sh.at[sl], out_h.at[c, sl])

    return k(dst2, ones_rows)


def _sc_build(degp, feat, prevp=None, b1r=None):
    n_pad = degp.shape[1]
    first_layer = prevp is None
    rmax = max(RB0, RB1)

    extra_in = () if first_layer else (prevp, b1r)
    extra_scratch = () if first_layer else (
        pltpu.VMEM((rmax, LANES), jnp.float32),
        pltpu.VMEM((rmax, LANES), jnp.float32),
        pltpu.VMEM((LANES,), jnp.float32),
    )

    @functools.partial(
        pl.kernel,
        out_type=jax.ShapeDtypeStruct((n_pad, LANES), jnp.float32),
        mesh=_sc_mesh(),
        compiler_params=_SC_PARAMS,
        scratch_types=[
            pltpu.VMEM((rmax, LANES), jnp.float32),
            pltpu.VMEM((rmax, LANES), jnp.float32),
            pltpu.VMEM((rmax, LANES), jnp.float32),
            *extra_scratch,
        ],
    )
    def k(degp_h, feat_h, *rest):
        if first_layer:
            (tbl_h, d0_v, d1_v, f_v) = rest
            p0_v = p1_v = b_v = prevp_h = b1_h = None
        else:
            (prevp_h, b1_h, tbl_h, d0_v, d1_v, f_v, p0_v, p1_v, b_v) = rest
        c = lax.axis_index("c")

        def build(rows_n, base):
            sl = pl.ds(base, rows_n)
            vs = pl.ds(0, rows_n)
            pltpu.sync_copy(degp_h.at[0, sl], d0_v.at[vs])
            pltpu.sync_copy(degp_h.at[1, sl], d1_v.at[vs])
            pltpu.sync_copy(feat_h.at[sl], f_v.at[vs])
            if not first_layer:
                pltpu.sync_copy(prevp_h.at[0, sl], p0_v.at[vs])
                pltpu.sync_copy(prevp_h.at[1, sl], p1_v.at[vs])
                pltpu.sync_copy(b1_h, b_v)

            @pl.loop(0, rows_n)
            def _(r):
                deg = d0_v[r] + d1_v[r] - 1.0
                dis = _rsqrt16(deg)
                if first_layer:
                    val = f_v[r] * dis
                else:
                    agg = p0_v[r] + p1_v[r] + f_v[r]
                    z = agg * dis + b_v[...]
                    val = jnp.maximum(z, 0.0) * dis
                f_v[r] = val

            pltpu.sync_copy(f_v.at[vs], tbl_h.at[sl])

        _core_split(c, build, RB0, RB1)

    return k(degp, feat, *extra_in)


def _sc_aggregate(src2, dst2, tbl):
    n_pad = tbl.shape[0]
    rows_per_sub = n_pad // NUM_SUB

    @functools.partial(
        pl.kernel,
        out_type=jax.ShapeDtypeStruct((NUM_CORES, n_pad, LANES), jnp.float32),
        mesh=_sc_mesh(),
        compiler_params=_SC_PARAMS,
        scratch_types=[
            pltpu.VMEM((max(CH0, CH1), CHUNK), jnp.int32),
            pltpu.VMEM((max(CH0, CH1), CHUNK), jnp.int32),
            pltpu.VMEM((CHUNK, LANES), jnp.float32),
            pltpu.VMEM((CHUNK, LANES), jnp.float32),
            pltpu.VMEM((rows_per_sub, LANES), jnp.float32),
            pltpu.VMEM_SHARED((n_pad, LANES), jnp.float32),
            pltpu.SemaphoreType.DMA,
            pltpu.SemaphoreType.DMA,
        ],
    )
    def k(src_h, dst_h, tbl_h, out_h, si_v, di_v, rows0, rows1, z_v, acc_sh,
          sem0, sem1):
        c = lax.axis_index("c")
        s = lax.axis_index("s")
        sl = pl.ds(s * rows_per_sub, rows_per_sub)

        @pl.loop(0, rows_per_sub)
        def _(r):
            z_v[r] = jnp.zeros((LANES,), jnp.float32)

        pltpu.sync_copy(z_v, acc_sh.at[sl])
        plsc.subcore_barrier()

        def run_edges(ch_n, base):
            pltpu.sync_copy(src_h.at[pl.ds(base, ch_n)],
                            si_v.at[pl.ds(0, ch_n)])
            pltpu.sync_copy(dst_h.at[pl.ds(base, ch_n)],
                            di_v.at[pl.ds(0, ch_n)])

            pltpu.async_copy(tbl_h.at[si_v.at[0]], rows0, sem0)

            @pl.when(ch_n > 1)
            def _():
                pltpu.async_copy(tbl_h.at[si_v.at[1]], rows1, sem1)

            @pl.loop(0, ch_n, step=2)
            def _(j):
                pltpu.make_async_copy(tbl_h.at[si_v.at[j]], rows0,
                                      sem0).wait()
                pltpu.sync_copy(rows0, acc_sh.at[di_v.at[j]], add=True)

                @pl.when(j + 2 < ch_n)
                def _():
                    pltpu.async_copy(tbl_h.at[si_v.at[j + 2]], rows0, sem0)

                @pl.when(j + 1 < ch_n)
                def _():
                    pltpu.make_async_copy(tbl_h.at[si_v.at[j + 1]], rows1,
                                          sem1).wait()
                    pltpu.sync_copy(rows1, acc_sh.at[di_v.at[j + 1]],
                                    add=True)

                    @pl.when(j + 3 < ch_n)
                    def _():
                        pltpu.async_copy(tbl_h.at[si_v.at[j + 3]], rows1,
                                         sem1)

        _core_split(c, run_edges, CH0, CH1)

        plsc.subcore_barrier()
        pltpu.sync_copy(acc_sh.at[sl], out_h.at[c, sl])

    return k(src2, dst2, tbl)


def _tc_mm1(x, w1, n_pad):
    d_in = x.shape[1]
    blk = 2560

    def body(x_ref, w_ref, h_ref):
        h_ref[...] = jnp.dot(x_ref[...], w_ref[...],
                             preferred_element_type=jnp.float32)

    return pl.pallas_call(
        body,
        grid=(n_pad // blk,),
        in_specs=[
            pl.BlockSpec((blk, d_in), lambda i: (i, 0)),
            pl.BlockSpec((d_in, LANES), lambda i: (0, 0)),
        ],
        out_specs=pl.BlockSpec((blk, LANES), lambda i: (i, 0)),
        out_shape=jax.ShapeDtypeStruct((n_pad, LANES), jnp.float32),
    )(x, w1)


def _tc_post(degp, aggp, tbl, w2, b2r, n):
    d_out = w2.shape[1]
    blk = 2000

    def body(d_ref, a_ref, t_ref, w_ref, b_ref, o_ref):
        deg = d_ref[0] + d_ref[1] - 1.0
        dis = lax.rsqrt(deg)
        gsc = (a_ref[0] + a_ref[1] + t_ref[...]) * dis
        o_ref[...] = (
            jnp.dot(gsc, w_ref[...], preferred_element_type=jnp.float32)
            + b_ref[...]
        )

    return pl.pallas_call(
        body,
        grid=(n // blk,),
        in_specs=[
            pl.BlockSpec((NUM_CORES, blk, LANES), lambda i: (0, i, 0)),
            pl.BlockSpec((NUM_CORES, blk, LANES), lambda i: (0, i, 0)),
            pl.BlockSpec((blk, LANES), lambda i: (i, 0)),
            pl.BlockSpec((LANES, d_out), lambda i: (0, 0)),
            pl.BlockSpec((1, d_out), lambda i: (0, 0)),
        ],
        out_specs=pl.BlockSpec((blk, d_out), lambda i: (i, 0)),
        out_shape=jax.ShapeDtypeStruct((n, d_out), jnp.float32),
    )(degp, aggp, tbl, w2, b2r)


def kernel(x, edge_index, W1, b1, W2, b2):
    n, _ = x.shape
    e = edge_index.shape[1]

    n_pad = ((n + NUM_TILES * NUM_SUB - 1) // (NUM_TILES * NUM_SUB)) * (
        NUM_TILES * NUM_SUB
    )
    per_pair = (CH0 + CH1) * CHUNK
    e_pad = ((e + NUM_SUB * per_pair - 1) // (NUM_SUB * per_pair)) * (
        NUM_SUB * per_pair
    )
    n_chunks = e_pad // CHUNK

    ei2 = jnp.pad(
        edge_index, ((0, 0), (0, e_pad - e)), constant_values=n_pad - 1
    ).reshape(2, n_chunks, CHUNK)
    src2 = ei2[0]
    dst2 = ei2[1]

    ones_rows = jnp.ones((n_pad // NUM_SUB, LANES), jnp.float32)

    degp = _sc_degree(dst2, ones_rows, n_pad)
    h1 = _tc_mm1(x, W1, n_pad)
    tbl1 = _sc_build(degp, h1)
    aggp1 = _sc_aggregate(src2, dst2, tbl1)
    tbl2 = _sc_build(degp, tbl1, prevp=aggp1, b1r=b1)
    aggp2 = _sc_aggregate(src2, dst2, tbl2)
    out = _tc_post(degp, aggp2, tbl2, W2, b2.reshape(1, -1), n)
    return out


# speedup vs baseline: 47.9355x; 1.3405x over previous
import functools

import jax
import jax.numpy as jnp
from jax import lax
from jax.experimental import pallas as pl
from jax.experimental.pallas import tpu as pltpu
from jax.experimental.pallas import tpu_sc as plsc

NUM_CORES = 2
NUM_SUB = 16
NUM_TILES = NUM_CORES * NUM_SUB
LANES = 16
CHUNK = 512
CH0 = 10
CH1 = 10


def _sc_mesh():
    return plsc.VectorSubcoreMesh(core_axis_name="c", subcore_axis_name="s")


_SC_PARAMS = pltpu.CompilerParams(
    use_tc_tiling_on_sc=False, needs_layout_passes=False
)


def _rsqrt16(x):
    i = plsc.bitcast(x, jnp.int32)
    y = plsc.bitcast(jnp.int32(0x5F3759DF) - jnp.right_shift(i, 1),
                     jnp.float32)
    for _ in range(3):
        y = y * (1.5 - 0.5 * x * y * y)
    return y


def _sc_degree(dst2, ones_rows, n_pad):
    rows_per_sub = n_pad // NUM_SUB

    @functools.partial(
        pl.kernel,
        out_type=jax.ShapeDtypeStruct((NUM_CORES, n_pad, LANES), jnp.float32),
        mesh=_sc_mesh(),
        compiler_params=_SC_PARAMS,
        scratch_types=[
            pltpu.VMEM((max(CH0, CH1), CHUNK), jnp.int32),
            pltpu.VMEM((rows_per_sub, LANES), jnp.float32),
            pltpu.VMEM_SHARED((n_pad, LANES), jnp.float32),
        ],
    )
    def k(dst_h, ones_h, out_h, di_v, ones_v, acc_sh):
        c = lax.axis_index("c")
        s = lax.axis_index("s")
        sl = pl.ds(s * rows_per_sub, rows_per_sub)
        pltpu.sync_copy(ones_h, ones_v)
        pltpu.sync_copy(ones_v, acc_sh.at[sl])
        plsc.subcore_barrier()

        def scatter_ones(ch_n, base):
            pltpu.sync_copy(dst_h.at[pl.ds(base, ch_n)],
                            di_v.at[pl.ds(0, ch_n)])

            @pl.loop(0, ch_n)
            def _(j):
                pltpu.sync_copy(ones_v.at[pl.ds(0, CHUNK)],
                                acc_sh.at[di_v.at[j]], add=True)

        @pl.when(c == 0)
        def _():
            scatter_ones(CH0, s * CH0)

        @pl.when(c == 1)
        def _():
            scatter_ones(CH1, NUM_SUB * CH0 + s * CH1)

        plsc.subcore_barrier()
        pltpu.sync_copy(acc_sh.at[sl], out_h.at[c, sl])

    return k(dst2, ones_rows)


def _sc_aggregate(src2, dst2, degp, feat, prevp=None, b1r=None):
    n_pad = degp.shape[1]
    rows_per_sub = n_pad // NUM_SUB
    first_layer = prevp is None

    extra_in = () if first_layer else (prevp, b1r)
    extra_scratch = () if first_layer else (
        pltpu.VMEM((rows_per_sub, LANES), jnp.float32),
        pltpu.VMEM((rows_per_sub, LANES), jnp.float32),
        pltpu.VMEM((LANES,), jnp.float32),
    )

    @functools.partial(
        pl.kernel,
        out_type=(
            jax.ShapeDtypeStruct((NUM_CORES, n_pad, LANES), jnp.float32),
            jax.ShapeDtypeStruct((n_pad, LANES), jnp.float32),
        ),
        mesh=_sc_mesh(),
        compiler_params=_SC_PARAMS,
        scratch_types=[
            pltpu.VMEM((max(CH0, CH1), CHUNK), jnp.int32),
            pltpu.VMEM((max(CH0, CH1), CHUNK), jnp.int32),
            pltpu.VMEM((CHUNK, LANES), jnp.float32),
            pltpu.VMEM((CHUNK, LANES), jnp.float32),
            pltpu.VMEM((rows_per_sub, LANES), jnp.float32),
            pltpu.VMEM((rows_per_sub, LANES), jnp.float32),
            pltpu.VMEM((rows_per_sub, LANES), jnp.float32),
            *extra_scratch,
            pltpu.VMEM_SHARED((n_pad, LANES), jnp.float32),
            pltpu.SemaphoreType.DMA,
            pltpu.SemaphoreType.DMA,
        ],
    )
    def k(src_h, dst_h, degp_h, feat_h, *rest):
        if first_layer:
            (out_h, tbl_h, si_v, di_v, rows0, rows1, d0_v, d1_v, f_v,
             acc_sh, sem0, sem1) = rest
            p0_v = p1_v = b_v = prevp_h = b1_h = None
        else:
            (prevp_h, b1_h, out_h, tbl_h, si_v, di_v, rows0, rows1, d0_v,
             d1_v, f_v, p0_v, p1_v, b_v, acc_sh, sem0, sem1) = rest
        c = lax.axis_index("c")
        s = lax.axis_index("s")
        sl = pl.ds(s * rows_per_sub, rows_per_sub)

        pltpu.sync_copy(degp_h.at[0, sl], d0_v)
        pltpu.sync_copy(degp_h.at[1, sl], d1_v)
        pltpu.sync_copy(feat_h.at[sl], f_v)
        if not first_layer:
            pltpu.sync_copy(prevp_h.at[0, sl], p0_v)
            pltpu.sync_copy(prevp_h.at[1, sl], p1_v)
            pltpu.sync_copy(b1_h, b_v)

        @pl.loop(0, rows_per_sub)
        def _(r):
            deg = d0_v[r] + d1_v[r] - 1.0
            dis = _rsqrt16(deg)
            if first_layer:
                val = f_v[r] * dis
            else:
                agg = p0_v[r] + p1_v[r] - f_v[r]
                z = agg * dis + b_v[...]
                val = jnp.maximum(z, 0.0) * dis
            f_v[r] = val

        pltpu.sync_copy(f_v, tbl_h.at[sl])
        pltpu.sync_copy(f_v, acc_sh.at[sl])
        plsc.subcore_barrier()

        def run_edges(ch_n, base):
            pltpu.sync_copy(src_h.at[pl.ds(base, ch_n)],
                            si_v.at[pl.ds(0, ch_n)])
            pltpu.sync_copy(dst_h.at[pl.ds(base, ch_n)],
                            di_v.at[pl.ds(0, ch_n)])

            pltpu.async_copy(tbl_h.at[si_v.at[0]], rows0, sem0)

            @pl.when(ch_n > 1)
            def _():
                pltpu.async_copy(tbl_h.at[si_v.at[1]], rows1, sem1)

            @pl.loop(0, ch_n, step=2)
            def _(j):
                pltpu.make_async_copy(tbl_h.at[si_v.at[j]], rows0,
                                      sem0).wait()
                pltpu.sync_copy(rows0, acc_sh.at[di_v.at[j]], add=True)

                @pl.when(j + 2 < ch_n)
                def _():
                    pltpu.async_copy(tbl_h.at[si_v.at[j + 2]], rows0, sem0)

                @pl.when(j + 1 < ch_n)
                def _():
                    pltpu.make_async_copy(tbl_h.at[si_v.at[j + 1]], rows1,
                                          sem1).wait()
                    pltpu.sync_copy(rows1, acc_sh.at[di_v.at[j + 1]],
                                    add=True)

                    @pl.when(j + 3 < ch_n)
                    def _():
                        pltpu.async_copy(tbl_h.at[si_v.at[j + 3]], rows1,
                                         sem1)

        @pl.when(c == 0)
        def _():
            run_edges(CH0, s * CH0)

        @pl.when(c == 1)
        def _():
            run_edges(CH1, NUM_SUB * CH0 + s * CH1)

        plsc.subcore_barrier()
        pltpu.sync_copy(acc_sh.at[sl], out_h.at[c, sl])

    return k(src2, dst2, degp, feat, *extra_in)


def _tc_mm1(x, w1, n_pad):
    d_in = x.shape[1]
    blk = 2560

    def body(x_ref, w_ref, h_ref):
        h_ref[...] = jnp.dot(x_ref[...], w_ref[...],
                             preferred_element_type=jnp.float32)

    return pl.pallas_call(
        body,
        grid=(n_pad // blk,),
        in_specs=[
            pl.BlockSpec((blk, d_in), lambda i: (i, 0)),
            pl.BlockSpec((d_in, LANES), lambda i: (0, 0)),
        ],
        out_specs=pl.BlockSpec((blk, LANES), lambda i: (i, 0)),
        out_shape=jax.ShapeDtypeStruct((n_pad, LANES), jnp.float32),
    )(x, w1)


def _tc_post(degp, aggp, tbl, w2, b2r, n):
    d_out = w2.shape[1]
    blk = 2000

    def body(d_ref, a_ref, t_ref, w_ref, b_ref, o_ref):
        deg = d_ref[0] + d_ref[1] - 1.0
        dis = lax.rsqrt(deg)
        gsc = (a_ref[0] + a_ref[1] - t_ref[...]) * dis
        o_ref[...] = (
            jnp.dot(gsc, w_ref[...], preferred_element_type=jnp.float32)
            + b_ref[...]
        )

    return pl.pallas_call(
        body,
        grid=(n // blk,),
        in_specs=[
            pl.BlockSpec((NUM_CORES, blk, LANES), lambda i: (0, i, 0)),
            pl.BlockSpec((NUM_CORES, blk, LANES), lambda i: (0, i, 0)),
            pl.BlockSpec((blk, LANES), lambda i: (i, 0)),
            pl.BlockSpec((LANES, d_out), lambda i: (0, 0)),
            pl.BlockSpec((1, d_out), lambda i: (0, 0)),
        ],
        out_specs=pl.BlockSpec((blk, d_out), lambda i: (i, 0)),
        out_shape=jax.ShapeDtypeStruct((n, d_out), jnp.float32),
    )(degp, aggp, tbl, w2, b2r)


def kernel(x, edge_index, W1, b1, W2, b2):
    n, _ = x.shape
    e = edge_index.shape[1]

    n_pad = ((n + NUM_TILES * NUM_SUB - 1) // (NUM_TILES * NUM_SUB)) * (
        NUM_TILES * NUM_SUB
    )
    per_pair = (CH0 + CH1) * CHUNK
    e_pad = ((e + NUM_SUB * per_pair - 1) // (NUM_SUB * per_pair)) * (
        NUM_SUB * per_pair
    )
    n_chunks = e_pad // CHUNK

    pad_idx = n + jnp.arange(e_pad - e, dtype=edge_index.dtype) % (n_pad - n)
    ei2 = jnp.concatenate(
        [edge_index, jnp.broadcast_to(pad_idx, (2, e_pad - e))], axis=1
    ).reshape(2, n_chunks, CHUNK)
    src2 = ei2[0]
    dst2 = ei2[1]

    ones_rows = jnp.ones((n_pad // NUM_SUB, LANES), jnp.float32)

    degp = _sc_degree(dst2, ones_rows, n_pad)
    h1 = _tc_mm1(x, W1, n_pad)
    aggp1, tbl1 = _sc_aggregate(src2, dst2, degp, h1)
    aggp2, tbl2 = _sc_aggregate(
        src2, dst2, degp, tbl1, prevp=aggp1, b1r=b1
    )
    out = _tc_post(degp, aggp2, tbl2, W2, b2.reshape(1, -1), n)
    return out


# speedup vs baseline: 48.0781x; 1.0030x over previous
import functools

import jax
import jax.numpy as jnp
from jax import lax
from jax.experimental import pallas as pl
from jax.experimental.pallas import tpu as pltpu
from jax.experimental.pallas import tpu_sc as plsc

NUM_CORES = 2
NUM_SUB = 16
NUM_TILES = NUM_CORES * NUM_SUB
LANES = 16
CHUNK = 512
CH0 = 10
CH1 = 10


def _sc_mesh():
    return plsc.VectorSubcoreMesh(core_axis_name="c", subcore_axis_name="s")


_SC_PARAMS = pltpu.CompilerParams(
    use_tc_tiling_on_sc=False, needs_layout_passes=False
)


def _rsqrt16(x):
    i = plsc.bitcast(x, jnp.int32)
    y = plsc.bitcast(jnp.int32(0x5F3759DF) - jnp.right_shift(i, 1),
                     jnp.float32)
    for _ in range(3):
        y = y * (1.5 - 0.5 * x * y * y)
    return y


def _sc_degree(dst2, ones_rows, n_pad):
    rows_per_sub = n_pad // NUM_SUB

    @functools.partial(
        pl.kernel,
        out_type=jax.ShapeDtypeStruct((NUM_CORES, n_pad, LANES), jnp.float32),
        mesh=_sc_mesh(),
        compiler_params=_SC_PARAMS,
        scratch_types=[
            pltpu.VMEM((max(CH0, CH1), CHUNK), jnp.int32),
            pltpu.VMEM((rows_per_sub, LANES), jnp.float32),
            pltpu.VMEM_SHARED((n_pad, LANES), jnp.float32),
            pltpu.SemaphoreType.DMA,
        ],
    )
    def k(dst_h, ones_h, out_h, di_v, ones_v, acc_sh, sem):
        c = lax.axis_index("c")
        s = lax.axis_index("s")
        sl = pl.ds(s * rows_per_sub, rows_per_sub)
        pltpu.sync_copy(ones_h, ones_v)
        pltpu.sync_copy(ones_v, acc_sh.at[sl])
        plsc.subcore_barrier()

        def scatter_ones(ch_n, base):
            pltpu.sync_copy(dst_h.at[pl.ds(base, ch_n)],
                            di_v.at[pl.ds(0, ch_n)])

            @pl.loop(0, ch_n)
            def _(j):
                pltpu.async_copy(ones_v.at[pl.ds(0, CHUNK)],
                                 acc_sh.at[di_v.at[j]], sem, add=True)

            @pl.loop(0, ch_n)
            def _(j):
                pltpu.make_async_copy(ones_v.at[pl.ds(0, CHUNK)],
                                      acc_sh.at[di_v.at[j]], sem).wait()

        @pl.when(c == 0)
        def _():
            scatter_ones(CH0, s * CH0)

        @pl.when(c == 1)
        def _():
            scatter_ones(CH1, NUM_SUB * CH0 + s * CH1)

        plsc.subcore_barrier()
        pltpu.sync_copy(acc_sh.at[sl], out_h.at[c, sl])

    return k(dst2, ones_rows)


def _sc_aggregate(src2, dst2, degp, feat, prevp=None, b1r=None):
    n_pad = degp.shape[1]
    rows_per_sub = n_pad // NUM_SUB
    first_layer = prevp is None

    extra_in = () if first_layer else (prevp, b1r)
    extra_scratch = () if first_layer else (
        pltpu.VMEM((rows_per_sub, LANES), jnp.float32),
        pltpu.VMEM((rows_per_sub, LANES), jnp.float32),
        pltpu.VMEM((LANES,), jnp.float32),
    )

    @functools.partial(
        pl.kernel,
        out_type=(
            jax.ShapeDtypeStruct((NUM_CORES, n_pad, LANES), jnp.float32),
            jax.ShapeDtypeStruct((n_pad, LANES), jnp.float32),
        ),
        mesh=_sc_mesh(),
        compiler_params=_SC_PARAMS,
        scratch_types=[
            pltpu.VMEM((max(CH0, CH1), CHUNK), jnp.int32),
            pltpu.VMEM((max(CH0, CH1), CHUNK), jnp.int32),
            pltpu.VMEM((CHUNK, LANES), jnp.float32),
            pltpu.VMEM((CHUNK, LANES), jnp.float32),
            pltpu.VMEM((rows_per_sub, LANES), jnp.float32),
            pltpu.VMEM((rows_per_sub, LANES), jnp.float32),
            pltpu.VMEM((rows_per_sub, LANES), jnp.float32),
            *extra_scratch,
            pltpu.VMEM_SHARED((n_pad, LANES), jnp.float32),
            pltpu.SemaphoreType.DMA,
            pltpu.SemaphoreType.DMA,
        ],
    )
    def k(src_h, dst_h, degp_h, feat_h, *rest):
        if first_layer:
            (out_h, tbl_h, si_v, di_v, rows0, rows1, d0_v, d1_v, f_v,
             acc_sh, sem0, sem1) = rest
            p0_v = p1_v = b_v = prevp_h = b1_h = None
        else:
            (prevp_h, b1_h, out_h, tbl_h, si_v, di_v, rows0, rows1, d0_v,
             d1_v, f_v, p0_v, p1_v, b_v, acc_sh, sem0, sem1) = rest
        c = lax.axis_index("c")
        s = lax.axis_index("s")
        sl = pl.ds(s * rows_per_sub, rows_per_sub)

        pltpu.sync_copy(degp_h.at[0, sl], d0_v)
        pltpu.sync_copy(degp_h.at[1, sl], d1_v)
        pltpu.sync_copy(feat_h.at[sl], f_v)
        if not first_layer:
            pltpu.sync_copy(prevp_h.at[0, sl], p0_v)
            pltpu.sync_copy(prevp_h.at[1, sl], p1_v)
            pltpu.sync_copy(b1_h, b_v)

        @pl.loop(0, rows_per_sub)
        def _(r):
            deg = d0_v[r] + d1_v[r] - 1.0
            dis = _rsqrt16(deg)
            if first_layer:
                val = f_v[r] * dis
            else:
                agg = p0_v[r] + p1_v[r] - f_v[r]
                z = agg * dis + b_v[...]
                val = jnp.maximum(z, 0.0) * dis
            f_v[r] = val

        pltpu.sync_copy(f_v, tbl_h.at[sl])
        pltpu.sync_copy(f_v, acc_sh.at[sl])
        plsc.subcore_barrier()

        def run_edges(ch_n, base):
            pltpu.sync_copy(src_h.at[pl.ds(base, ch_n)],
                            si_v.at[pl.ds(0, ch_n)])
            pltpu.sync_copy(dst_h.at[pl.ds(base, ch_n)],
                            di_v.at[pl.ds(0, ch_n)])

            pltpu.async_copy(tbl_h.at[si_v.at[0]], rows0, sem0)

            @pl.when(ch_n > 1)
            def _():
                pltpu.async_copy(tbl_h.at[si_v.at[1]], rows1, sem1)

            @pl.loop(0, ch_n, step=2)
            def _(j):
                pltpu.make_async_copy(tbl_h.at[si_v.at[j]], rows0,
                                      sem0).wait()
                pltpu.sync_copy(rows0, acc_sh.at[di_v.at[j]], add=True)

                @pl.when(j + 2 < ch_n)
                def _():
                    pltpu.async_copy(tbl_h.at[si_v.at[j + 2]], rows0, sem0)

                @pl.when(j + 1 < ch_n)
                def _():
                    pltpu.make_async_copy(tbl_h.at[si_v.at[j + 1]], rows1,
                                          sem1).wait()
                    pltpu.sync_copy(rows1, acc_sh.at[di_v.at[j + 1]],
                                    add=True)

                    @pl.when(j + 3 < ch_n)
                    def _():
                        pltpu.async_copy(tbl_h.at[si_v.at[j + 3]], rows1,
                                         sem1)

        @pl.when(c == 0)
        def _():
            run_edges(CH0, s * CH0)

        @pl.when(c == 1)
        def _():
            run_edges(CH1, NUM_SUB * CH0 + s * CH1)

        plsc.subcore_barrier()
        pltpu.sync_copy(acc_sh.at[sl], out_h.at[c, sl])

    return k(src2, dst2, degp, feat, *extra_in)


def _tc_mm1(x, w1, n_pad):
    d_in = x.shape[1]
    blk = 2560

    def body(x_ref, w_ref, h_ref):
        h_ref[...] = jnp.dot(x_ref[...], w_ref[...],
                             preferred_element_type=jnp.float32)

    return pl.pallas_call(
        body,
        grid=(n_pad // blk,),
        in_specs=[
            pl.BlockSpec((blk, d_in), lambda i: (i, 0)),
            pl.BlockSpec((d_in, LANES), lambda i: (0, 0)),
        ],
        out_specs=pl.BlockSpec((blk, LANES), lambda i: (i, 0)),
        out_shape=jax.ShapeDtypeStruct((n_pad, LANES), jnp.float32),
    )(x, w1)


def _tc_post(degp, aggp, tbl, w2, b2r, n):
    d_out = w2.shape[1]
    blk = 2000

    def body(d_ref, a_ref, t_ref, w_ref, b_ref, o_ref):
        deg = d_ref[0] + d_ref[1] - 1.0
        dis = lax.rsqrt(deg)
        gsc = (a_ref[0] + a_ref[1] - t_ref[...]) * dis
        o_ref[...] = (
            jnp.dot(gsc, w_ref[...], preferred_element_type=jnp.float32)
            + b_ref[...]
        )

    return pl.pallas_call(
        body,
        grid=(n // blk,),
        in_specs=[
            pl.BlockSpec((NUM_CORES, blk, LANES), lambda i: (0, i, 0)),
            pl.BlockSpec((NUM_CORES, blk, LANES), lambda i: (0, i, 0)),
            pl.BlockSpec((blk, LANES), lambda i: (i, 0)),
            pl.BlockSpec((LANES, d_out), lambda i: (0, 0)),
            pl.BlockSpec((1, d_out), lambda i: (0, 0)),
        ],
        out_specs=pl.BlockSpec((blk, d_out), lambda i: (i, 0)),
        out_shape=jax.ShapeDtypeStruct((n, d_out), jnp.float32),
    )(degp, aggp, tbl, w2, b2r)


def kernel(x, edge_index, W1, b1, W2, b2):
    n, _ = x.shape
    e = edge_index.shape[1]

    n_pad = ((n + NUM_TILES * NUM_SUB - 1) // (NUM_TILES * NUM_SUB)) * (
        NUM_TILES * NUM_SUB
    )
    per_pair = (CH0 + CH1) * CHUNK
    e_pad = ((e + NUM_SUB * per_pair - 1) // (NUM_SUB * per_pair)) * (
        NUM_SUB * per_pair
    )
    n_chunks = e_pad // CHUNK

    pad_idx = n + jnp.arange(e_pad - e, dtype=edge_index.dtype) % (n_pad - n)
    ei2 = jnp.concatenate(
        [edge_index, jnp.broadcast_to(pad_idx, (2, e_pad - e))], axis=1
    ).reshape(2, n_chunks, CHUNK)
    src2 = ei2[0]
    dst2 = ei2[1]

    ones_rows = jnp.ones((n_pad // NUM_SUB, LANES), jnp.float32)

    degp = _sc_degree(dst2, ones_rows, n_pad)
    h1 = _tc_mm1(x, W1, n_pad)
    aggp1, tbl1 = _sc_aggregate(src2, dst2, degp, h1)
    aggp2, tbl2 = _sc_aggregate(
        src2, dst2, degp, tbl1, prevp=aggp1, b1r=b1
    )
    out = _tc_post(degp, aggp2, tbl2, W2, b2.reshape(1, -1), n)
    return out
